# baseline (speedup 1.0000x reference)
import functools

import jax
import jax.numpy as jnp
from jax.experimental import pallas as pl
from jax.experimental.pallas import tpu as pltpu

_BB = 256
_NB = 4
_KB = 2048


def _mv_kernel(x0, x1, x2, x3, w_ref, b_ref, o_ref, a0, a1, a2, a3,
               *, vocab, nk):
    k = pl.program_id(0)
    xs = (x0, x1, x2, x3)
    accs = (a0, a1, a2, a3)

    @pl.when(k == 0)
    def _init():
        for a in accs:
            a[...] = jnp.zeros_like(a)

    @pl.when(k < nk - 1)
    def _body():
        wc = w_ref[...]
        for x, a in zip(xs, accs):
            a[...] += x[...] * wc

    @pl.when(k == nk - 1)
    def _tail():
        col = jax.lax.broadcasted_iota(jnp.int32, (1, _KB), 1)
        valid = col + k * _KB < vocab
        wc = w_ref[...]
        for j, (x, a) in enumerate(zip(xs, accs)):
            a[...] += jnp.where(valid, x[...], 0.0) * wc
            o_ref[pl.ds(j * _BB, _BB), :] = (
                jnp.sum(a[...], axis=1, keepdims=True) + b_ref[0, 0]
            )


@jax.jit
def kernel(x, weight, bias):
    batch, vocab = x.shape
    nk = pl.cdiv(vocab, _KB)
    wpad = jnp.pad(weight.reshape(-1), (0, nk * _KB - vocab))

    def band_spec(j):
        return pl.BlockSpec((_BB, _KB), lambda k, j=j: (j, k))

    out = pl.pallas_call(
        functools.partial(_mv_kernel, vocab=vocab, nk=nk),
        grid=(nk,),
        in_specs=[band_spec(j) for j in range(_NB)] + [
            pl.BlockSpec((1, _KB), lambda k: (0, k)),
            pl.BlockSpec((1, 1), lambda k: (0, 0)),
        ],
        out_specs=pl.BlockSpec((batch, 1), lambda k: (0, 0)),
        out_shape=jax.ShapeDtypeStruct((batch, 1), jnp.float32),
        scratch_shapes=[pltpu.VMEM((_BB, _KB), jnp.float32)
                        for _ in range(_NB)],
        compiler_params=pltpu.CompilerParams(
            dimension_semantics=("arbitrary",)
        ),
    )(x, x, x, x, wpad.reshape(1, -1), bias.reshape(1, 1))
    return out


# speedup vs baseline: 1.0173x; 1.0173x over previous
import jax
import jax.numpy as jnp
from jax.experimental import pallas as pl
from jax.experimental.pallas import tpu as pltpu

_RB = 64


def _mv_kernel(x_ref, w_ref, b_ref, o_ref):
    o_ref[...] = (
        jnp.sum(x_ref[...] * w_ref[...], axis=1, keepdims=True) + b_ref[0, 0]
    )


@jax.jit
def kernel(x, weight, bias):
    batch, vocab = x.shape
    out = pl.pallas_call(
        _mv_kernel,
        grid=(batch // _RB,),
        in_specs=[
            pl.BlockSpec((_RB, vocab), lambda i: (i, 0)),
            pl.BlockSpec((1, vocab), lambda i: (0, 0)),
            pl.BlockSpec((1, 1), lambda i: (0, 0)),
        ],
        out_specs=pl.BlockSpec((_RB, 1), lambda i: (i, 0)),
        out_shape=jax.ShapeDtypeStruct((batch, 1), jnp.float32),
        compiler_params=pltpu.CompilerParams(
            dimension_semantics=("arbitrary",)
        ),
    )(x, weight.reshape(1, -1), bias.reshape(1, 1))
    return out
